# baseline (speedup 1.0000x reference)
import functools

import jax
import jax.numpy as jnp
from jax import lax
from jax.experimental import pallas as pl
from jax.experimental.pallas import tpu as pltpu
from jax.experimental.pallas import tpu_sc as plsc

_X_MAX = 100.0
_ALPHA = 0.75
_B = 4096
_NC = 2
_NS = 16
_NW = _NC * _NS
_BPW = _B // _NW
_E = 64
_L = 16


def _sc_gather_dot(w_data, v_data, w_embed, v_embed, w_bias, v_bias):
    mesh = plsc.VectorSubcoreMesh(core_axis_name="c", subcore_axis_name="s")

    @functools.partial(
        pl.kernel,
        mesh=mesh,
        out_type=(
            jax.ShapeDtypeStruct((_B,), jnp.float32),
            jax.ShapeDtypeStruct((_B,), jnp.float32),
        ),
        scratch_types=[
            pltpu.VMEM((_BPW,), jnp.int32),
            pltpu.VMEM((_BPW,), jnp.int32),
            pltpu.VMEM((_BPW, _E), jnp.float32),
            pltpu.VMEM((_BPW, _E), jnp.float32),
            pltpu.VMEM((_BPW, 1), jnp.float32),
            pltpu.VMEM((_BPW, 1), jnp.float32),
            pltpu.VMEM((_BPW,), jnp.float32),
            pltpu.VMEM((_BPW,), jnp.float32),
            pltpu.SemaphoreType.DMA,
        ],
    )
    def k(w_data_h, v_data_h, w_embed_h, v_embed_h, w_bias_h, v_bias_h,
          dot_out, s_out, widx, vidx, wrows, vrows, wb, vb, dotv, sv, sem):
        wid = lax.axis_index("s") * _NC + lax.axis_index("c")
        base = wid * _BPW
        pltpu.sync_copy(w_data_h.at[pl.ds(base, _BPW)], widx)
        pltpu.sync_copy(v_data_h.at[pl.ds(base, _BPW)], vidx)

        for g in range(_BPW // _L):
            wv = widx[pl.ds(g * _L, _L)]
            vv = vidx[pl.ds(g * _L, _L)]
            for r in range(_L):
                j = g * _L + r
                wr = wv[r]
                vr = vv[r]
                pltpu.async_copy(w_embed_h.at[pl.ds(wr, 1), :],
                                 wrows.at[pl.ds(j, 1), :], sem)
                pltpu.async_copy(v_embed_h.at[pl.ds(vr, 1), :],
                                 vrows.at[pl.ds(j, 1), :], sem)
                pltpu.async_copy(w_bias_h.at[pl.ds(wr, 1), :],
                                 wb.at[pl.ds(j, 1), :], sem)
                pltpu.async_copy(v_bias_h.at[pl.ds(vr, 1), :],
                                 vb.at[pl.ds(j, 1), :], sem)
        pltpu.make_async_copy(w_embed_h.at[pl.ds(0, _BPW), :], wrows, sem).wait()
        pltpu.make_async_copy(v_embed_h.at[pl.ds(0, _BPW), :], vrows, sem).wait()
        pltpu.make_async_copy(w_bias_h.at[pl.ds(0, _BPW), :], wb, sem).wait()
        pltpu.make_async_copy(v_bias_h.at[pl.ds(0, _BPW), :], vb, sem).wait()

        lane = lax.iota(jnp.int32, _L)
        dn = lax.GatherDimensionNumbers(
            offset_dims=(), collapsed_slice_dims=(0,), start_index_map=(0,))

        def perm(v, bit):
            idx = (lane ^ bit).reshape(_L, 1)
            return lax.gather(v, idx, dn, (1,),
                              mode=lax.GatherScatterMode.PROMISE_IN_BOUNDS)

        def merge(a, b, bit):
            hi = (lane & bit) != 0
            return jnp.where(hi, b, a) + perm(jnp.where(hi, a, b), bit)

        for g in range(_BPW // _L):
            vs = []
            svec = jnp.zeros((_L,), jnp.float32)
            for r in range(_L):
                j = g * _L + r
                acc = wrows[j, pl.ds(0, _L)] * vrows[j, pl.ds(0, _L)]
                for kk in range(1, _E // _L):
                    sl = pl.ds(kk * _L, _L)
                    acc = acc + wrows[j, sl] * vrows[j, sl]
                vs.append(acc)
                s_j = wb[j, pl.ds(0, 1)][0] + vb[j, pl.ds(0, 1)][0]
                svec = jnp.where(lane == r, s_j, svec)
            for bit in (1, 2, 4, 8):
                vs = [merge(vs[2 * i], vs[2 * i + 1], bit)
                      for i in range(len(vs) // 2)]
            dotv[pl.ds(g * _L, _L)] = vs[0]
            sv[pl.ds(g * _L, _L)] = svec

        pltpu.sync_copy(dotv, dot_out.at[pl.ds(base, _BPW)])
        pltpu.sync_copy(sv, s_out.at[pl.ds(base, _BPW)])

    return k(w_data, v_data, w_embed, v_embed, w_bias, v_bias)


def _tc_combine_body(dot_ref, s_ref, lab_ref, out_ref):
    d = dot_ref[...]
    s = s_ref[...]
    lab = lab_ref[...]
    w = jnp.minimum(jnp.exp(_ALPHA * jnp.log(lab * (1.0 / _X_MAX))), 1.0)
    a = d - jnp.log(lab)
    s1 = jnp.sum(w * a * a)
    s2 = jnp.sum(w * a)
    s3 = jnp.sum(w)
    s4 = jnp.sum(s)
    s5 = jnp.sum(s * s)
    bf = float(_B)
    out_ref[0, 0] = (bf * s1 + 2.0 * s2 * s4 + s3 * s5) / (bf * bf)


def _tc_combine(dot, s, labels):
    return pl.pallas_call(
        _tc_combine_body,
        out_shape=jax.ShapeDtypeStruct((1, 1), jnp.float32),
        out_specs=pl.BlockSpec(memory_space=pltpu.SMEM),
    )(dot.reshape(32, 128), s.reshape(32, 128), labels.reshape(32, 128))


def kernel(w_data, v_data, labels, w_embed, w_bias, v_embed, v_bias):
    dot, s = _sc_gather_dot(
        w_data.astype(jnp.int32), v_data.astype(jnp.int32),
        w_embed, v_embed, w_bias, v_bias,
    )
    out = _tc_combine(dot, s, labels)
    return out[0, 0]


# speedup vs baseline: 1.0497x; 1.0497x over previous
import functools

import jax
import jax.numpy as jnp
from jax import lax
from jax.experimental import pallas as pl
from jax.experimental.pallas import tpu as pltpu
from jax.experimental.pallas import tpu_sc as plsc

_X_MAX = 100.0
_ALPHA = 0.75
_B = 4096
_NC = 2
_NS = 16
_NW = _NC * _NS
_BPW = _B // _NW
_E = 64
_L = 16

_DN = lax.GatherDimensionNumbers(
    offset_dims=(), collapsed_slice_dims=(0,), start_index_map=(0,))


def _perm(v, bit, lane):
    idx = (lane ^ bit).reshape(_L, 1)
    return lax.gather(v, idx, _DN, (1,),
                      mode=lax.GatherScatterMode.PROMISE_IN_BOUNDS)


def _merge(a, b, bit, lane):
    hi = (lane & bit) != 0
    return jnp.where(hi, b, a) + _perm(jnp.where(hi, a, b), bit, lane)


def _sc_embed_dot(w_data, v_data, w_embed2, v_embed2):
    mesh = plsc.VectorSubcoreMesh(core_axis_name="c", subcore_axis_name="s")

    @functools.partial(
        pl.kernel,
        mesh=mesh,
        out_type=jax.ShapeDtypeStruct((_B,), jnp.float32),
        scratch_types=[
            pltpu.VMEM((_BPW,), jnp.int32),
            pltpu.VMEM((_BPW,), jnp.int32),
            pltpu.VMEM((_BPW,), jnp.int32),
            pltpu.VMEM((_BPW,), jnp.int32),
            pltpu.VMEM((_BPW, 2 * _E), jnp.float32),
            pltpu.VMEM((_BPW, 2 * _E), jnp.float32),
            pltpu.VMEM((_BPW,), jnp.float32),
            pltpu.SemaphoreType.DMA,
        ],
    )
    def k(w_data_h, v_data_h, w_embed_h, v_embed_h,
          dot_out, widx, vidx, widx2, vidx2, wrows, vrows, dotv, sem):
        wid = lax.axis_index("s") * _NC + lax.axis_index("c")
        base = wid * _BPW
        pltpu.sync_copy(w_data_h.at[pl.ds(base, _BPW)], widx)
        pltpu.sync_copy(v_data_h.at[pl.ds(base, _BPW)], vidx)
        for g in range(_BPW // _L):
            sl = pl.ds(g * _L, _L)
            widx2[sl] = widx[sl] >> 1
            vidx2[sl] = vidx[sl] >> 1
        c1 = pltpu.async_copy(w_embed_h.at[widx2], wrows, sem)
        c2 = pltpu.async_copy(v_embed_h.at[vidx2], vrows, sem)
        c1.wait()
        c2.wait()

        lane = lax.iota(jnp.int32, _L)
        for g in range(_BPW // _L):
            wv = widx[pl.ds(g * _L, _L)]
            vv = vidx[pl.ds(g * _L, _L)]
            wofs = (wv & 1) << 6
            vofs = (vv & 1) << 6
            vs = []
            for r in range(_L):
                j = g * _L + r
                wo = pl.multiple_of(wofs[r], 64)
                vo = pl.multiple_of(vofs[r], 64)
                acc = wrows[j, pl.ds(wo, _L)] * vrows[j, pl.ds(vo, _L)]
                for kk in range(1, _E // _L):
                    acc = acc + (wrows[j, pl.ds(wo + kk * _L, _L)]
                                 * vrows[j, pl.ds(vo + kk * _L, _L)])
                vs.append(acc)
            for bit in (1, 2, 4, 8):
                vs = [_merge(vs[2 * i], vs[2 * i + 1], bit, lane)
                      for i in range(len(vs) // 2)]
            dotv[pl.ds(g * _L, _L)] = vs[0]

        pltpu.sync_copy(dotv, dot_out.at[pl.ds(base, _BPW)])

    return k(w_data, v_data, w_embed2, v_embed2)


def _sc_bias_sum(w_data, v_data, w_bias, v_bias):
    mesh = plsc.VectorSubcoreMesh(core_axis_name="c", subcore_axis_name="s")

    @functools.partial(
        pl.kernel,
        mesh=mesh,
        out_type=jax.ShapeDtypeStruct((_B,), jnp.float32),
        scratch_types=[
            pltpu.VMEM((_BPW,), jnp.int32),
            pltpu.VMEM((_BPW,), jnp.int32),
            pltpu.VMEM((_BPW * 8 + 16,), jnp.float32),
            pltpu.VMEM((_BPW * 8 + 16,), jnp.float32),
            pltpu.VMEM((_BPW,), jnp.float32),
            pltpu.SemaphoreType.DMA,
        ],
    )
    def k(w_data_h, v_data_h, w_bias_h, v_bias_h,
          s_out, widx, vidx, wwin, vwin, sv, sem):
        wid = lax.axis_index("s") * _NC + lax.axis_index("c")
        base = wid * _BPW
        pltpu.sync_copy(w_data_h.at[pl.ds(base, _BPW)], widx)
        pltpu.sync_copy(v_data_h.at[pl.ds(base, _BPW)], vidx)

        for g in range(_BPW // _L):
            wv = widx[pl.ds(g * _L, _L)]
            vv = vidx[pl.ds(g * _L, _L)]
            wbse = (wv >> 3) << 3
            vbse = (vv >> 3) << 3
            for r in range(_L):
                j = g * _L + r
                rw = pl.multiple_of(wbse[r], 8)
                rv = pl.multiple_of(vbse[r], 8)
                pltpu.async_copy(w_bias_h.at[pl.ds(rw, 8)],
                                 wwin.at[pl.ds(j * 8, 8)], sem)
                pltpu.async_copy(v_bias_h.at[pl.ds(rv, 8)],
                                 vwin.at[pl.ds(j * 8, 8)], sem)
        pltpu.make_async_copy(w_bias_h.at[pl.ds(0, _BPW * 8)],
                              wwin.at[pl.ds(0, _BPW * 8)], sem).wait()
        pltpu.make_async_copy(v_bias_h.at[pl.ds(0, _BPW * 8)],
                              vwin.at[pl.ds(0, _BPW * 8)], sem).wait()

        lane = lax.iota(jnp.int32, _L)
        zero = jnp.zeros((_L,), jnp.float32)
        for g in range(_BPW // _L):
            wv = widx[pl.ds(g * _L, _L)] & 7
            vv = vidx[pl.ds(g * _L, _L)] & 7
            vs = []
            for r in range(_L):
                j = g * _L + r
                ww = wwin[pl.ds(j * 8, _L)]
                vw = vwin[pl.ds(j * 8, _L)]
                m = (jnp.where(lane == wv[r], ww, zero)
                     + jnp.where(lane == vv[r], vw, zero))
                vs.append(m)
            for bit in (1, 2, 4, 8):
                vs = [_merge(vs[2 * i], vs[2 * i + 1], bit, lane)
                      for i in range(len(vs) // 2)]
            sv[pl.ds(g * _L, _L)] = vs[0]

        pltpu.sync_copy(sv, s_out.at[pl.ds(base, _BPW)])

    return k(w_data, v_data, w_bias, v_bias)


def _tc_combine_body(dot_ref, s_ref, lab_ref, out_ref):
    d = dot_ref[...]
    s = s_ref[...]
    lab = lab_ref[...]
    w = jnp.minimum(jnp.exp(_ALPHA * jnp.log(lab * (1.0 / _X_MAX))), 1.0)
    a = d - jnp.log(lab)
    s1 = jnp.sum(w * a * a)
    s2 = jnp.sum(w * a)
    s3 = jnp.sum(w)
    s4 = jnp.sum(s)
    s5 = jnp.sum(s * s)
    bf = float(_B)
    out_ref[0, 0] = (bf * s1 + 2.0 * s2 * s4 + s3 * s5) / (bf * bf)


def _tc_combine(dot, s, labels):
    return pl.pallas_call(
        _tc_combine_body,
        out_shape=jax.ShapeDtypeStruct((1, 1), jnp.float32),
        out_specs=pl.BlockSpec(memory_space=pltpu.SMEM),
    )(dot.reshape(32, 128), s.reshape(32, 128), labels.reshape(32, 128))


def kernel(w_data, v_data, labels, w_embed, w_bias, v_embed, v_bias):
    wi = w_data.astype(jnp.int32)
    vi = v_data.astype(jnp.int32)
    dot = _sc_embed_dot(wi, vi,
                        w_embed.reshape(50000, 2 * _E),
                        v_embed.reshape(50000, 2 * _E))
    s = _sc_bias_sum(wi, vi, w_bias.reshape(-1), v_bias.reshape(-1))
    out = _tc_combine(dot, s, labels)
    return out[0, 0]


# speedup vs baseline: 1.1158x; 1.0630x over previous
import functools

import jax
import jax.numpy as jnp
from jax import lax
from jax.experimental import pallas as pl
from jax.experimental.pallas import tpu as pltpu
from jax.experimental.pallas import tpu_sc as plsc

_X_MAX = 100.0
_ALPHA = 0.75
_B = 4096
_NC = 2
_NS = 16
_NW = _NC * _NS
_BPW = _B // _NW
_E = 64
_L = 16

_DN = lax.GatherDimensionNumbers(
    offset_dims=(), collapsed_slice_dims=(0,), start_index_map=(0,))


def _perm(v, bit, lane):
    idx = (lane ^ bit).reshape(_L, 1)
    return lax.gather(v, idx, _DN, (1,),
                      mode=lax.GatherScatterMode.PROMISE_IN_BOUNDS)


def _merge(a, b, bit, lane):
    hi = (lane & bit) != 0
    return jnp.where(hi, b, a) + _perm(jnp.where(hi, a, b), bit, lane)


def _sc_embed_dot(w_data, v_data, w_embed2, v_embed2):
    mesh = plsc.VectorSubcoreMesh(core_axis_name="c", subcore_axis_name="s")

    @functools.partial(
        pl.kernel,
        mesh=mesh,
        out_type=jax.ShapeDtypeStruct((_B,), jnp.float32),
        scratch_types=[
            pltpu.VMEM((_BPW,), jnp.int32),
            pltpu.VMEM((_BPW,), jnp.int32),
            pltpu.VMEM((_BPW,), jnp.int32),
            pltpu.VMEM((_BPW,), jnp.int32),
            pltpu.VMEM((_BPW, 2 * _E), jnp.float32),
            pltpu.VMEM((_BPW, 2 * _E), jnp.float32),
            pltpu.VMEM((_BPW,), jnp.float32),
            pltpu.SemaphoreType.DMA,
        ],
    )
    def k(w_data_h, v_data_h, w_embed_h, v_embed_h,
          dot_out, widx, vidx, widx2, vidx2, wrows, vrows, dotv, sem):
        wid = lax.axis_index("s") * _NC + lax.axis_index("c")
        base = wid * _BPW
        pltpu.sync_copy(w_data_h.at[pl.ds(base, _BPW)], widx)
        pltpu.sync_copy(v_data_h.at[pl.ds(base, _BPW)], vidx)
        c1 = pltpu.async_copy(w_embed_h.at[widx], wrows, sem)
        c2 = pltpu.async_copy(v_embed_h.at[vidx], vrows, sem)
        c1.wait()
        c2.wait()

        lane = lax.iota(jnp.int32, _L)
        for g in range(_BPW // _L):
            vs = []
            for r in range(_L):
                j = g * _L + r
                acc = wrows[j, pl.ds(0, _L)] * vrows[j, pl.ds(0, _L)]
                for kk in range(1, _E // _L):
                    sl = pl.ds(kk * _L, _L)
                    acc = acc + wrows[j, sl] * vrows[j, sl]
                vs.append(acc)
            for bit in (1, 2, 4, 8):
                vs = [_merge(vs[2 * i], vs[2 * i + 1], bit, lane)
                      for i in range(len(vs) // 2)]
            dotv[pl.ds(g * _L, _L)] = vs[0]

        pltpu.sync_copy(dotv, dot_out.at[pl.ds(base, _BPW)])

    return k(w_data, v_data, w_embed2, v_embed2)


def _sc_bias_sum(w_data, v_data, w_bias, v_bias):
    mesh = plsc.VectorSubcoreMesh(core_axis_name="c", subcore_axis_name="s")

    @functools.partial(
        pl.kernel,
        mesh=mesh,
        out_type=jax.ShapeDtypeStruct((_B,), jnp.float32),
        scratch_types=[
            pltpu.VMEM((_BPW,), jnp.int32),
            pltpu.VMEM((_BPW,), jnp.int32),
            pltpu.VMEM((_BPW * 8 + 16,), jnp.float32),
            pltpu.VMEM((_BPW * 8 + 16,), jnp.float32),
            pltpu.VMEM((_BPW,), jnp.float32),
            pltpu.SemaphoreType.DMA,
        ],
    )
    def k(w_data_h, v_data_h, w_bias_h, v_bias_h,
          s_out, widx, vidx, wwin, vwin, sv, sem):
        wid = lax.axis_index("s") * _NC + lax.axis_index("c")
        base = wid * _BPW
        pltpu.sync_copy(w_data_h.at[pl.ds(base, _BPW)], widx)
        pltpu.sync_copy(v_data_h.at[pl.ds(base, _BPW)], vidx)

        for g in range(_BPW // _L):
            wv = widx[pl.ds(g * _L, _L)]
            vv = vidx[pl.ds(g * _L, _L)]
            wbse = (wv >> 3) << 3
            vbse = (vv >> 3) << 3
            for r in range(_L):
                j = g * _L + r
                rw = pl.multiple_of(wbse[r], 8)
                rv = pl.multiple_of(vbse[r], 8)
                pltpu.async_copy(w_bias_h.at[pl.ds(rw, 8)],
                                 wwin.at[pl.ds(j * 8, 8)], sem)
                pltpu.async_copy(v_bias_h.at[pl.ds(rv, 8)],
                                 vwin.at[pl.ds(j * 8, 8)], sem)
        pltpu.make_async_copy(w_bias_h.at[pl.ds(0, _BPW * 8)],
                              wwin.at[pl.ds(0, _BPW * 8)], sem).wait()
        pltpu.make_async_copy(v_bias_h.at[pl.ds(0, _BPW * 8)],
                              vwin.at[pl.ds(0, _BPW * 8)], sem).wait()

        lane = lax.iota(jnp.int32, _L)
        zero = jnp.zeros((_L,), jnp.float32)
        for g in range(_BPW // _L):
            wv = widx[pl.ds(g * _L, _L)] & 7
            vv = vidx[pl.ds(g * _L, _L)] & 7
            vs = []
            for r in range(_L):
                j = g * _L + r
                ww = wwin[pl.ds(j * 8, _L)]
                vw = vwin[pl.ds(j * 8, _L)]
                m = (jnp.where(lane == wv[r], ww, zero)
                     + jnp.where(lane == vv[r], vw, zero))
                vs.append(m)
            for bit in (1, 2, 4, 8):
                vs = [_merge(vs[2 * i], vs[2 * i + 1], bit, lane)
                      for i in range(len(vs) // 2)]
            sv[pl.ds(g * _L, _L)] = vs[0]

        pltpu.sync_copy(sv, s_out.at[pl.ds(base, _BPW)])

    return k(w_data, v_data, w_bias, v_bias)


def _tc_combine_body(dot_ref, s_ref, lab_ref, out_ref):
    d = dot_ref[...]
    s = s_ref[...]
    lab = lab_ref[...]
    w = jnp.minimum(jnp.exp(_ALPHA * jnp.log(lab * (1.0 / _X_MAX))), 1.0)
    a = d - jnp.log(lab)
    s1 = jnp.sum(w * a * a)
    s2 = jnp.sum(w * a)
    s3 = jnp.sum(w)
    s4 = jnp.sum(s)
    s5 = jnp.sum(s * s)
    bf = float(_B)
    out_ref[0, 0] = (bf * s1 + 2.0 * s2 * s4 + s3 * s5) / (bf * bf)


def _tc_combine(dot, s, labels):
    return pl.pallas_call(
        _tc_combine_body,
        out_shape=jax.ShapeDtypeStruct((1, 1), jnp.float32),
        out_specs=pl.BlockSpec(memory_space=pltpu.SMEM),
    )(dot.reshape(32, 128), s.reshape(32, 128), labels.reshape(32, 128))


def kernel(w_data, v_data, labels, w_embed, w_bias, v_embed, v_bias):
    wi = w_data.astype(jnp.int32)
    vi = v_data.astype(jnp.int32)
    dot = _sc_embed_dot(wi, vi,
                        jnp.pad(w_embed, ((0, 0), (0, _E))),
                        jnp.pad(v_embed, ((0, 0), (0, _E))))
    s = _sc_bias_sum(wi, vi, w_bias.reshape(-1), v_bias.reshape(-1))
    out = _tc_combine(dot, s, labels)
    return out[0, 0]


# speedup vs baseline: 1.1187x; 1.0027x over previous
import functools

import jax
import jax.numpy as jnp
from jax import lax
from jax.experimental import pallas as pl
from jax.experimental.pallas import tpu as pltpu
from jax.experimental.pallas import tpu_sc as plsc

_X_MAX = 100.0
_ALPHA = 0.75
_B = 4096
_NC = 2
_NS = 16
_NW = _NC * _NS
_BPW = _B // _NW
_E = 64
_L = 16

_DN = lax.GatherDimensionNumbers(
    offset_dims=(), collapsed_slice_dims=(0,), start_index_map=(0,))


def _perm(v, bit, lane):
    idx = (lane ^ bit).reshape(_L, 1)
    return lax.gather(v, idx, _DN, (1,),
                      mode=lax.GatherScatterMode.PROMISE_IN_BOUNDS)


def _merge(a, b, bit, lane):
    hi = (lane & bit) != 0
    return jnp.where(hi, b, a) + _perm(jnp.where(hi, a, b), bit, lane)


def _sc_embed_dot(w_data, v_data, w_embed2, v_embed2):
    mesh = plsc.VectorSubcoreMesh(core_axis_name="c", subcore_axis_name="s")

    @functools.partial(
        pl.kernel,
        mesh=mesh,
        out_type=jax.ShapeDtypeStruct((_B,), jnp.float32),
        scratch_types=[
            pltpu.VMEM((_BPW,), jnp.int32),
            pltpu.VMEM((_BPW,), jnp.int32),
            pltpu.VMEM((_BPW, 2 * _E), jnp.float32),
            pltpu.VMEM((_BPW, 2 * _E), jnp.float32),
            pltpu.VMEM((_BPW,), jnp.float32),
            pltpu.SemaphoreType.DMA,
        ],
    )
    def k(w_data_h, v_data_h, w_embed_h, v_embed_h,
          dot_out, widx, vidx, wrows, vrows, dotv, sem):
        wid = lax.axis_index("s") * _NC + lax.axis_index("c")
        base = wid * _BPW
        pltpu.sync_copy(w_data_h.at[pl.ds(base, _BPW)], widx)
        pltpu.sync_copy(v_data_h.at[pl.ds(base, _BPW)], vidx)
        c1 = pltpu.async_copy(w_embed_h.at[widx], wrows, sem)
        c2 = pltpu.async_copy(v_embed_h.at[vidx], vrows, sem)
        c1.wait()
        c2.wait()

        lane = lax.iota(jnp.int32, _L)
        for g in range(_BPW // _L):
            vs = []
            for r in range(_L):
                j = g * _L + r
                acc = wrows[j, pl.ds(0, _L)] * vrows[j, pl.ds(0, _L)]
                for kk in range(1, _E // _L):
                    sl = pl.ds(kk * _L, _L)
                    acc = acc + wrows[j, sl] * vrows[j, sl]
                vs.append(acc)
            for bit in (1, 2, 4, 8):
                vs = [_merge(vs[2 * i], vs[2 * i + 1], bit, lane)
                      for i in range(len(vs) // 2)]
            dotv[pl.ds(g * _L, _L)] = vs[0]

        pltpu.sync_copy(dotv, dot_out.at[pl.ds(base, _BPW)])

    return k(w_data, v_data, w_embed2, v_embed2)


def _sc_bias_sum(w_data, v_data, w_bias, v_bias):
    mesh = plsc.VectorSubcoreMesh(core_axis_name="c", subcore_axis_name="s")

    @functools.partial(
        pl.kernel,
        mesh=mesh,
        out_type=jax.ShapeDtypeStruct((_B,), jnp.float32),
        scratch_types=[
            pltpu.VMEM((_BPW,), jnp.int32),
            pltpu.VMEM((_BPW,), jnp.int32),
            pltpu.VMEM((_BPW * 8 + 16,), jnp.float32),
            pltpu.VMEM((_BPW * 8 + 16,), jnp.float32),
            pltpu.VMEM((_BPW,), jnp.float32),
            pltpu.SemaphoreType.DMA,
        ],
    )
    def k(w_data_h, v_data_h, w_bias_h, v_bias_h,
          s_out, widx, vidx, wwin, vwin, sv, sem):
        wid = lax.axis_index("s") * _NC + lax.axis_index("c")
        base = wid * _BPW
        pltpu.sync_copy(w_data_h.at[pl.ds(base, _BPW)], widx)
        pltpu.sync_copy(v_data_h.at[pl.ds(base, _BPW)], vidx)

        for g in range(_BPW // _L):
            wv = widx[pl.ds(g * _L, _L)]
            vv = vidx[pl.ds(g * _L, _L)]
            wbse = (wv >> 3) << 3
            vbse = (vv >> 3) << 3
            for r in range(_L):
                j = g * _L + r
                rw = pl.multiple_of(wbse[r], 8)
                rv = pl.multiple_of(vbse[r], 8)
                pltpu.async_copy(w_bias_h.at[pl.ds(rw, 8)],
                                 wwin.at[pl.ds(j * 8, 8)], sem)
                pltpu.async_copy(v_bias_h.at[pl.ds(rv, 8)],
                                 vwin.at[pl.ds(j * 8, 8)], sem)
        pltpu.make_async_copy(w_bias_h.at[pl.ds(0, _BPW * 8)],
                              wwin.at[pl.ds(0, _BPW * 8)], sem).wait()
        pltpu.make_async_copy(v_bias_h.at[pl.ds(0, _BPW * 8)],
                              vwin.at[pl.ds(0, _BPW * 8)], sem).wait()

        lane = lax.iota(jnp.int32, _L)
        zero = jnp.zeros((_L,), jnp.float32)
        for g in range(_BPW // _L):
            wv = widx[pl.ds(g * _L, _L)] & 7
            vv = vidx[pl.ds(g * _L, _L)] & 7
            vs = []
            for r in range(_L):
                j = g * _L + r
                ww = wwin[pl.ds(j * 8, _L)]
                vw = vwin[pl.ds(j * 8, _L)]
                m = (jnp.where(lane == wv[r], ww, zero)
                     + jnp.where(lane == vv[r], vw, zero))
                vs.append(m)
            for bit in (1, 2, 4, 8):
                vs = [_merge(vs[2 * i], vs[2 * i + 1], bit, lane)
                      for i in range(len(vs) // 2)]
            sv[pl.ds(g * _L, _L)] = vs[0]

        pltpu.sync_copy(sv, s_out.at[pl.ds(base, _BPW)])

    return k(w_data, v_data, w_bias, v_bias)


def _tc_combine_body(dot_ref, s_ref, lab_ref, out_ref):
    d = dot_ref[...]
    s = s_ref[...]
    lab = lab_ref[...]
    w = jnp.minimum(jnp.exp(_ALPHA * jnp.log(lab * (1.0 / _X_MAX))), 1.0)
    a = d - jnp.log(lab)
    s1 = jnp.sum(w * a * a)
    s2 = jnp.sum(w * a)
    s3 = jnp.sum(w)
    s4 = jnp.sum(s)
    s5 = jnp.sum(s * s)
    bf = float(_B)
    out_ref[0, 0] = (bf * s1 + 2.0 * s2 * s4 + s3 * s5) / (bf * bf)


def _tc_combine(dot, s, labels):
    return pl.pallas_call(
        _tc_combine_body,
        out_shape=jax.ShapeDtypeStruct((1, 1), jnp.float32),
        out_specs=pl.BlockSpec(memory_space=pltpu.SMEM),
    )(dot.reshape(32, 128), s.reshape(32, 128), labels.reshape(32, 128))


def kernel(w_data, v_data, labels, w_embed, w_bias, v_embed, v_bias):
    wi = w_data.astype(jnp.int32)
    vi = v_data.astype(jnp.int32)
    dot = _sc_embed_dot(wi, vi,
                        jnp.pad(w_embed, ((0, 0), (0, _E))),
                        jnp.pad(v_embed, ((0, 0), (0, _E))))
    s = _sc_bias_sum(wi, vi, w_bias.reshape(-1), v_bias.reshape(-1))
    out = _tc_combine(dot, s, labels)
    return out[0, 0]
